# baseline (speedup 1.0000x reference)
import functools

import jax
import jax.numpy as jnp
from jax import lax
from jax.experimental import pallas as pl
from jax.experimental.pallas import tpu as pltpu
from jax.experimental.pallas import tpu_sc as plsc

_NC = 2
_NS = 16

_C = 1024
_BLK = 1024


def _lookup_body(step_hbm, emb_hbm, out_hbm, step_smem, row_spmem):
    cid = lax.axis_index("c")

    @pl.when(cid == 0)
    def _():
        pltpu.sync_copy(step_hbm, step_smem)
        s = step_smem[0]
        pltpu.sync_copy(emb_hbm.at[s], row_spmem)
        pltpu.sync_copy(row_spmem, out_hbm)


def _sc_lookup(step1, emb2):
    mesh = plsc.ScalarSubcoreMesh(axis_name="c", num_cores=_NC)
    run = functools.partial(
        pl.kernel,
        out_type=jax.ShapeDtypeStruct((_C,), jnp.float32),
        mesh=mesh,
        scratch_types=[
            pltpu.SMEM((1,), jnp.int32),
            pltpu.VMEM_SHARED((_C,), jnp.float32),
        ],
    )(_lookup_body)
    return run(step1, emb2).reshape(1, _C)


def _tc_body(x_ref, row_ref, out_ref):
    out_ref[...] = x_ref[...] + row_ref[...]


def _tc_add(x2, row):
    n_rows = x2.shape[0]
    return pl.pallas_call(
        _tc_body,
        grid=(n_rows // _BLK,),
        in_specs=[
            pl.BlockSpec((_BLK, _C), lambda i: (i, 0)),
            pl.BlockSpec((1, _C), lambda i: (0, 0)),
        ],
        out_specs=pl.BlockSpec((_BLK, _C), lambda i: (i, 0)),
        out_shape=jax.ShapeDtypeStruct((n_rows, _C), jnp.float32),
    )(x2, row)


def kernel(x_layer, step, step_embedding):
    B, S, C = x_layer.shape
    x2 = x_layer.reshape(B * S, C)
    emb2 = step_embedding.reshape(step_embedding.shape[0], C)
    step1 = jnp.asarray(step, jnp.int32).reshape(1)

    row = _sc_lookup(step1, emb2)
    out = _tc_add(x2, row)
    return out.reshape(B, S, C)
